# baseline (speedup 1.0000x reference)
import jax
import jax.numpy as jnp
from jax import lax
from jax.experimental import pallas as pl
from jax.experimental.pallas import tpu as pltpu
from jax.experimental.pallas import tpu_sc as plsc

f32 = jnp.float32
i32 = jnp.int32

N = 10000
E = 320000
F = 128
TW = 16
NC, NS = 2, 16
NW = NC * NS
EPW = E // NW
BB = 80
NB = EPW // BB
NPAD = 10240
ZROWS_PT = NPAD // NS

_MESH = plsc.VectorSubcoreMesh(
    core_axis_name="c", subcore_axis_name="s", num_cores=NC, num_subcores=NS)
_SC_PARAMS = pltpu.CompilerParams(
    needs_layout_passes=False, use_tc_tiling_on_sc=False)


def _blk(shape, imap):
    return pl.BlockSpec(shape, imap)


def _tc_prep(xin, W, Asrc, Adst):
    def body(x_r, w_r, as_r, ad_r, h_r, s_r, d_r):
        h = jnp.dot(x_r[...], w_r[...], preferred_element_type=f32)
        h_r[...] = h
        s_r[...] = jnp.dot(h, as_r[...], preferred_element_type=f32, precision=lax.Precision.HIGHEST)
        d_r[...] = jnp.dot(h, ad_r[...], preferred_element_type=f32, precision=lax.Precision.HIGHEST)
    nb = 10
    bn = N // nb
    return pl.pallas_call(
        body,
        grid=(nb,),
        in_specs=[_blk((bn, F), lambda i: (i, 0)),
                  _blk((F, F), lambda i: (0, 0)),
                  _blk((F, TW), lambda i: (0, 0)),
                  _blk((F, TW), lambda i: (0, 0))],
        out_specs=[_blk((bn, F), lambda i: (i, 0)),
                   _blk((bn, TW), lambda i: (i, 0)),
                   _blk((bn, TW), lambda i: (i, 0))],
        out_shape=[jax.ShapeDtypeStruct((N, F), f32),
                   jax.ShapeDtypeStruct((N, TW), f32),
                   jax.ShapeDtypeStruct((N, TW), f32)],
    )(xin, W, Asrc, Adst)


def _tc_mid(oparts, dparts, b1, W2, Asrc, Adst):
    nb = 10
    bn = N // nb

    def body(p_r, dp_r, b_r, w_r, as_r, ad_r, h2_r, s_r, d_r):
        raw = p_r[0] + p_r[1]
        den = dp_r[0] + dp_r[1]
        rd = 1.0 / (den[:, 0:8] + 1e-16)
        rde = jnp.reshape(
            jnp.broadcast_to(rd[:, :, None], (bn, 8, 16)), (bn, F))
        h = jax.nn.relu(raw * rde + b_r[...])
        h2 = jnp.dot(h, w_r[...], preferred_element_type=f32)
        h2_r[...] = h2
        s_r[...] = jnp.dot(h2, as_r[...], preferred_element_type=f32, precision=lax.Precision.HIGHEST)
        d_r[...] = jnp.dot(h2, ad_r[...], preferred_element_type=f32, precision=lax.Precision.HIGHEST)

    return pl.pallas_call(
        body,
        grid=(nb,),
        in_specs=[_blk((NC, bn, F), lambda i: (0, i, 0)),
                  _blk((NC, bn, TW), lambda i: (0, i, 0)),
                  _blk((1, F), lambda i: (0, 0)),
                  _blk((F, F), lambda i: (0, 0)),
                  _blk((F, TW), lambda i: (0, 0)),
                  _blk((F, TW), lambda i: (0, 0))],
        out_specs=[_blk((bn, F), lambda i: (i, 0)),
                   _blk((bn, TW), lambda i: (i, 0)),
                   _blk((bn, TW), lambda i: (i, 0))],
        out_shape=[jax.ShapeDtypeStruct((N, F), f32),
                   jax.ShapeDtypeStruct((N, TW), f32),
                   jax.ShapeDtypeStruct((N, TW), f32)],
    )(oparts, dparts, b1, W2, Asrc, Adst)


def _tc_final(oparts, dparts, b2, Wc1, bc1, Wc2p, bc2p):
    nb = 10
    bn = N // nb

    def body(p_r, dp_r, b_r, w1_r, b1_r, w2_r, b2_r, emb_r, lg_r):
        raw = p_r[0] + p_r[1]
        den = dp_r[0] + dp_r[1]
        rd = 1.0 / (den[:, 0:1] + 1e-16)
        emb = raw * jnp.broadcast_to(rd, (bn, F)) + b_r[...]
        emb_r[...] = emb
        hc = jax.nn.relu(jnp.dot(emb, w1_r[...], preferred_element_type=f32)
                         + b1_r[...])
        lg_r[...] = jnp.dot(hc, w2_r[...], preferred_element_type=f32) + b2_r[...]

    return pl.pallas_call(
        body,
        grid=(nb,),
        in_specs=[_blk((NC, bn, F), lambda i: (0, i, 0)),
                  _blk((NC, bn, TW), lambda i: (0, i, 0)),
                  _blk((1, F), lambda i: (0, 0)),
                  _blk((F, F), lambda i: (0, 0)),
                  _blk((1, F), lambda i: (0, 0)),
                  _blk((F, F), lambda i: (0, 0)),
                  _blk((1, F), lambda i: (0, 0))],
        out_specs=[_blk((bn, F), lambda i: (i, 0)),
                   _blk((bn, F), lambda i: (i, 0))],
        out_shape=[jax.ShapeDtypeStruct((N, F), f32),
                   jax.ShapeDtypeStruct((N, F), f32)],
    )(oparts, dparts, b2, Wc1, bc1, Wc2p, bc2p)


def _sc_edge(src, dst, S, D, h, Z8, Z128, hs):
    def body(src_r, dst_r, s_r, d_r, h_r, z8_r, z128_r, dp_r, op_r,
             si0, di0, dw0, sg0, dg0, ex0, hb0,
             si1, di1, dw1, sg1, dg1, ex1, hb1,
             is0, is1, i2s0, i2s1, gs0, gs1, ws0, ws1, dsh, osh):
        tid = lax.axis_index("s")
        cid = lax.axis_index("c")
        wid = cid * NS + tid
        ebase = wid * EPW
        sets = ((si0, di0, dw0, sg0, dg0, ex0, hb0, is0, i2s0, gs0, ws0),
                (si1, di1, dw1, sg1, dg1, ex1, hb1, is1, i2s1, gs1, ws1))

        def zinit(c, carry):
            r0 = tid * ZROWS_PT + c * BB
            pltpu.sync_copy(z8_r.at[pl.ds(r0, BB)], ex0)
            pltpu.sync_copy(ex0, dsh.at[pl.ds(r0, BB)])
            pltpu.sync_copy(z128_r.at[pl.ds(r0, BB)], hb0)
            pltpu.sync_copy(hb0, osh.at[pl.ds(r0, BB)])
            return carry

        lax.fori_loop(0, ZROWS_PT // BB, zinit, 0)
        pltpu.sync_copy(z8_r.at[pl.ds(0, BB)], ex0)
        pltpu.sync_copy(z8_r.at[pl.ds(0, BB)], ex1)
        plsc.subcore_barrier()

        rowpat = lax.iota(i32, 16) >> 3
        cols8 = lax.iota(i32, 16) & 7
        zeros16 = jnp.zeros((16,), i32)
        hcols = cols8 if hs == 1 else zeros16

        def i1_cps(b, s):
            base = ebase + b * BB
            return (pltpu.make_async_copy(src_r.at[pl.ds(base, BB)], s[0], s[7]),
                    pltpu.make_async_copy(dst_r.at[pl.ds(base, BB)], s[1], s[7]))

        def i2_cps(b, s):
            base = ebase + b * BB
            return (pltpu.make_async_copy(dst_r.at[pl.ds(base, BB)], s[2], s[8]),)

        def g_cps(s):
            return (pltpu.make_async_copy(s_r.at[s[0]], s[3], s[9]),
                    pltpu.make_async_copy(d_r.at[s[1]], s[4], s[9]),
                    pltpu.make_async_copy(h_r.at[s[0]], s[6], s[9]))

        def w_start(s):
            pltpu.async_copy(s[5], dsh.at[s[2]], s[10], add=True)
            pltpu.async_copy(s[6], osh.at[s[2]], s[10], add=True)

        def w_wait(s):
            pltpu.make_async_copy(s[5], dsh.at[s[2]], s[10]).wait()
            pltpu.make_async_copy(s[6], osh.at[s[2]], s[10]).wait()

        def start(cps):
            for c in cps:
                c.start()

        def wait(cps):
            for c in cps:
                c.wait()

        def compute(s):
            sg, dg, exb, hb = s[3], s[4], s[5], s[6]

            def pair(k, c2):
                rows = rowpat + 2 * k
                a = (plsc.load_gather(sg, [rows, cols8])
                     + plsc.load_gather(dg, [rows, cols8]))
                a = jnp.where(a > 0, a, 0.2 * a)
                plsc.store_scatter(exb, [rows, cols8], jnp.exp(a))
                return c2

            lax.fori_loop(0, BB // 2, pair, 0, unroll=8)

            def edge(e, c2):
                esplat = zeros16 + e
                wrow = plsc.load_gather(exb, [esplat, hcols])
                for j in range(8):
                    if hs == 1:
                        wj = jnp.take_along_axis(wrow, zeros16 + j, axis=0,
                                                 mode="promise_in_bounds")
                    else:
                        wj = wrow
                    hb[e, pl.ds(j * 16, 16)] = hb[e, pl.ds(j * 16, 16)] * wj
                return c2

            lax.fori_loop(0, BB, edge, 0, unroll=8)

        for b, s in ((0, sets[0]), (1, sets[1])):
            start(i1_cps(b, s))
            start(i2_cps(b, s))
        for b, s in ((0, sets[0]), (1, sets[1])):
            wait(i1_cps(b, s))
            start(g_cps(s))

        def iter_pair(t, carry):
            a = 2 * t
            for off in (0, 1):
                s = sets[off]
                b = a + off
                nxt = b + 2
                wait(g_cps(s))

                @pl.when(nxt < NB)
                def _():
                    start(i1_cps(nxt, s))

                compute(s)
                wait(i2_cps(b, s))
                w_start(s)
            for off in (0, 1):
                s = sets[off]
                b = a + off
                nxt = b + 2
                w_wait(s)

                @pl.when(nxt < NB)
                def _():
                    start(i2_cps(nxt, s))
                    wait(i1_cps(nxt, s))
                    start(g_cps(s))

            return carry

        lax.fori_loop(0, NB // 2, iter_pair, 0)

        sA = sets[0]
        wait(g_cps(sA))
        compute(sA)
        wait(i2_cps(NB - 1, sA))
        w_start(sA)
        w_wait(sA)
        plsc.subcore_barrier()

        def export(c, carry):
            r0 = tid * ZROWS_PT + c * BB
            pltpu.sync_copy(dsh.at[pl.ds(r0, BB)], ex0)
            pltpu.sync_copy(ex0, dp_r.at[cid, pl.ds(r0, BB)])
            pltpu.sync_copy(osh.at[pl.ds(r0, BB)], hb0)
            pltpu.sync_copy(hb0, op_r.at[cid, pl.ds(r0, BB)])
            return carry

        lax.fori_loop(0, ZROWS_PT // BB, export, 0)

    call = pl.kernel(
        body,
        out_type=(jax.ShapeDtypeStruct((NC, NPAD, TW), f32),
                  jax.ShapeDtypeStruct((NC, NPAD, F), f32)),
        mesh=_MESH,
        compiler_params=_SC_PARAMS,
        scratch_types=[
            pltpu.VMEM((BB,), i32), pltpu.VMEM((BB,), i32),
            pltpu.VMEM((BB,), i32),
            pltpu.VMEM((BB, TW), f32), pltpu.VMEM((BB, TW), f32),
            pltpu.VMEM((BB, TW), f32), pltpu.VMEM((BB, F), f32),
            pltpu.VMEM((BB,), i32), pltpu.VMEM((BB,), i32),
            pltpu.VMEM((BB,), i32),
            pltpu.VMEM((BB, TW), f32), pltpu.VMEM((BB, TW), f32),
            pltpu.VMEM((BB, TW), f32), pltpu.VMEM((BB, F), f32),
            pltpu.SemaphoreType.DMA, pltpu.SemaphoreType.DMA,
            pltpu.SemaphoreType.DMA, pltpu.SemaphoreType.DMA,
            pltpu.SemaphoreType.DMA, pltpu.SemaphoreType.DMA,
            pltpu.SemaphoreType.DMA, pltpu.SemaphoreType.DMA,
            pltpu.VMEM_SHARED((NPAD, TW), f32),
            pltpu.VMEM_SHARED((NPAD, F), f32),
        ],
    )
    return call(src, dst, S, D, h, Z8, Z128)


def kernel(x, edge_index, W1, att_src1, att_dst1, b1,
           W2, att_src2, att_dst2, b2, Wc1, bc1, Wc2, bc2):
    src = edge_index[0]
    dst = edge_index[1]

    a1s = att_src1.reshape(8, 16)
    a1d = att_dst1.reshape(8, 16)
    eye8 = jnp.eye(8, dtype=f32)
    As1 = jnp.pad(jnp.einsum("hc,hg->hcg", a1s, eye8).reshape(F, 8),
                  ((0, 0), (0, TW - 8)))
    Ad1 = jnp.pad(jnp.einsum("hc,hg->hcg", a1d, eye8).reshape(F, 8),
                  ((0, 0), (0, TW - 8)))
    As2 = jnp.pad(att_src2.reshape(F, 1), ((0, 0), (0, TW - 1)))
    Ad2 = jnp.pad(att_dst2.reshape(F, 1), ((0, 0), (0, TW - 1)))

    Z8 = jnp.zeros((NPAD, TW), f32)
    Z128 = jnp.zeros((NPAD, F), f32)
    b1r = b1.reshape(1, F)
    b2r = b2.reshape(1, F)
    bc1r = bc1.reshape(1, F)
    Wc2p = jnp.pad(Wc2, ((0, 0), (0, F - 2)))
    bc2p = jnp.pad(bc2, (0, F - 2)).reshape(1, F)

    h1, S1, D1 = _tc_prep(x, W1, As1, Ad1)
    dp1, op1 = _sc_edge(src, dst, S1, D1, h1, Z8, Z128, hs=1)
    h2, S2, D2 = _tc_mid(op1, dp1, b1r, W2, As2, Ad2)
    dp2, op2 = _sc_edge(src, dst, S2, D2, h2, Z8, Z128, hs=0)
    emb, logitsp = _tc_final(op2, dp2, b2r, Wc1, bc1r, Wc2p, bc2p)
    return emb, logitsp[:, :2]


# speedup vs baseline: 1.3234x; 1.3234x over previous
import jax
import jax.numpy as jnp
from jax import lax
from jax.experimental import pallas as pl
from jax.experimental.pallas import tpu as pltpu
from jax.experimental.pallas import tpu_sc as plsc

f32 = jnp.float32
i32 = jnp.int32

N = 10000
E = 320000
F = 128
TW = 16
NC, NS = 2, 16
NW = NC * NS
EPW = E // NW
BB = 80
NB = EPW // BB
NPAD = 10240
ZROWS_PT = NPAD // NS

_MESH = plsc.VectorSubcoreMesh(
    core_axis_name="c", subcore_axis_name="s", num_cores=NC, num_subcores=NS)
_SC_PARAMS = pltpu.CompilerParams(
    needs_layout_passes=False, use_tc_tiling_on_sc=False)


def _blk(shape, imap):
    return pl.BlockSpec(shape, imap)


def _tc_prep(xin, W, Asrc, Adst):
    def body(x_r, w_r, as_r, ad_r, h_r, s_r, d_r):
        h = jnp.dot(x_r[...], w_r[...], preferred_element_type=f32)
        h_r[...] = h
        s_r[...] = jnp.dot(h, as_r[...], preferred_element_type=f32, precision=lax.Precision.HIGHEST)
        d_r[...] = jnp.dot(h, ad_r[...], preferred_element_type=f32, precision=lax.Precision.HIGHEST)
    nb = 10
    bn = N // nb
    return pl.pallas_call(
        body,
        grid=(nb,),
        in_specs=[_blk((bn, F), lambda i: (i, 0)),
                  _blk((F, F), lambda i: (0, 0)),
                  _blk((F, TW), lambda i: (0, 0)),
                  _blk((F, TW), lambda i: (0, 0))],
        out_specs=[_blk((bn, F), lambda i: (i, 0)),
                   _blk((bn, TW), lambda i: (i, 0)),
                   _blk((bn, TW), lambda i: (i, 0))],
        out_shape=[jax.ShapeDtypeStruct((N, F), f32),
                   jax.ShapeDtypeStruct((N, TW), f32),
                   jax.ShapeDtypeStruct((N, TW), f32)],
    )(xin, W, Asrc, Adst)


def _tc_mid(oparts, dparts, b1, W2, Asrc, Adst):
    nb = 10
    bn = N // nb

    def body(p_r, dp_r, b_r, w_r, as_r, ad_r, h2_r, s_r, d_r):
        raw = p_r[0] + p_r[1]
        den = dp_r[0] + dp_r[1]
        rd = 1.0 / (den[:, 0:8] + 1e-16)
        rde = jnp.reshape(
            jnp.broadcast_to(rd[:, :, None], (bn, 8, 16)), (bn, F))
        h = jax.nn.relu(raw * rde + b_r[...])
        h2 = jnp.dot(h, w_r[...], preferred_element_type=f32)
        h2_r[...] = h2
        s_r[...] = jnp.dot(h2, as_r[...], preferred_element_type=f32, precision=lax.Precision.HIGHEST)
        d_r[...] = jnp.dot(h2, ad_r[...], preferred_element_type=f32, precision=lax.Precision.HIGHEST)

    return pl.pallas_call(
        body,
        grid=(nb,),
        in_specs=[_blk((NC, bn, F), lambda i: (0, i, 0)),
                  _blk((NC, bn, TW), lambda i: (0, i, 0)),
                  _blk((1, F), lambda i: (0, 0)),
                  _blk((F, F), lambda i: (0, 0)),
                  _blk((F, TW), lambda i: (0, 0)),
                  _blk((F, TW), lambda i: (0, 0))],
        out_specs=[_blk((bn, F), lambda i: (i, 0)),
                   _blk((bn, TW), lambda i: (i, 0)),
                   _blk((bn, TW), lambda i: (i, 0))],
        out_shape=[jax.ShapeDtypeStruct((N, F), f32),
                   jax.ShapeDtypeStruct((N, TW), f32),
                   jax.ShapeDtypeStruct((N, TW), f32)],
    )(oparts, dparts, b1, W2, Asrc, Adst)


def _tc_final(oparts, dparts, b2, Wc1, bc1, Wc2p, bc2p):
    nb = 10
    bn = N // nb

    def body(p_r, dp_r, b_r, w1_r, b1_r, w2_r, b2_r, emb_r, lg_r):
        raw = p_r[0] + p_r[1]
        den = dp_r[0] + dp_r[1]
        rd = 1.0 / (den[:, 0:1] + 1e-16)
        emb = raw * jnp.broadcast_to(rd, (bn, F)) + b_r[...]
        emb_r[...] = emb
        hc = jax.nn.relu(jnp.dot(emb, w1_r[...], preferred_element_type=f32)
                         + b1_r[...])
        lg_r[...] = jnp.dot(hc, w2_r[...], preferred_element_type=f32) + b2_r[...]

    return pl.pallas_call(
        body,
        grid=(nb,),
        in_specs=[_blk((NC, bn, F), lambda i: (0, i, 0)),
                  _blk((NC, bn, TW), lambda i: (0, i, 0)),
                  _blk((1, F), lambda i: (0, 0)),
                  _blk((F, F), lambda i: (0, 0)),
                  _blk((1, F), lambda i: (0, 0)),
                  _blk((F, F), lambda i: (0, 0)),
                  _blk((1, F), lambda i: (0, 0))],
        out_specs=[_blk((bn, F), lambda i: (i, 0)),
                   _blk((bn, F), lambda i: (i, 0))],
        out_shape=[jax.ShapeDtypeStruct((N, F), f32),
                   jax.ShapeDtypeStruct((N, F), f32)],
    )(oparts, dparts, b2, Wc1, bc1, Wc2p, bc2p)


def _sc_edge(src, dst, S, D, h, Z8, Z128, hs):
    def body(src_r, dst_r, s_r, d_r, h_r, z8_r, z128_r, dp_r, op_r,
             si0, di0, dw0, sg0, dg0, ex0, hb0,
             si1, di1, dw1, sg1, dg1, ex1, hb1,
             is0, is1, i2s0, i2s1, gs0, gs1, ws0, ws1, dsh, osh):
        tid = lax.axis_index("s")
        cid = lax.axis_index("c")
        wid = cid * NS + tid
        ebase = wid * EPW
        sets = ((si0, di0, dw0, sg0, dg0, ex0, hb0, is0, i2s0, gs0, ws0),
                (si1, di1, dw1, sg1, dg1, ex1, hb1, is1, i2s1, gs1, ws1))

        def zinit(c, carry):
            r0 = tid * ZROWS_PT + c * BB
            pltpu.sync_copy(z8_r.at[pl.ds(r0, BB)], ex0)
            pltpu.sync_copy(ex0, dsh.at[pl.ds(r0, BB)])
            pltpu.sync_copy(z128_r.at[pl.ds(r0, BB)], hb0)
            pltpu.sync_copy(hb0, osh.at[pl.ds(r0, BB)])
            return carry

        lax.fori_loop(0, ZROWS_PT // BB, zinit, 0)
        pltpu.sync_copy(z8_r.at[pl.ds(0, BB)], ex0)
        pltpu.sync_copy(z8_r.at[pl.ds(0, BB)], ex1)
        plsc.subcore_barrier()

        rowpat = lax.iota(i32, 16) >> 3
        cols8 = lax.iota(i32, 16) & 7
        zeros16 = jnp.zeros((16,), i32)
        hcols = cols8 if hs == 1 else zeros16

        def i1_cps(b, s):
            base = ebase + b * BB
            return (pltpu.make_async_copy(src_r.at[pl.ds(base, BB)], s[0], s[7]),
                    pltpu.make_async_copy(dst_r.at[pl.ds(base, BB)], s[1], s[7]))

        def i2_cps(b, s):
            base = ebase + b * BB
            return (pltpu.make_async_copy(dst_r.at[pl.ds(base, BB)], s[2], s[8]),)

        def g_cps(s):
            return (pltpu.make_async_copy(s_r.at[s[0]], s[3], s[9]),
                    pltpu.make_async_copy(d_r.at[s[1]], s[4], s[9]),
                    pltpu.make_async_copy(h_r.at[s[0]], s[6], s[9]))

        def w_start(s):
            pltpu.async_copy(s[5], dsh.at[s[2]], s[10], add=True)
            pltpu.async_copy(s[6], osh.at[s[2]], s[10], add=True)

        def w_wait(s):
            pltpu.make_async_copy(s[5], dsh.at[s[2]], s[10]).wait()
            pltpu.make_async_copy(s[6], osh.at[s[2]], s[10]).wait()

        def start(cps):
            for c in cps:
                c.start()

        def wait(cps):
            for c in cps:
                c.wait()

        def compute(s):
            sg, dg, exb, hb = s[3], s[4], s[5], s[6]

            def pair(k, c2):
                rows = rowpat + 2 * k
                a = (plsc.load_gather(sg, [rows, cols8])
                     + plsc.load_gather(dg, [rows, cols8]))
                a = jnp.where(a > 0, a, 0.2 * a)
                plsc.store_scatter(exb, [rows, cols8], jnp.exp(a))
                return c2

            lax.fori_loop(0, BB // 2, pair, 0, unroll=8)

            def edge(e, c2):
                esplat = zeros16 + e
                wrow = plsc.load_gather(exb, [esplat, hcols])
                for j in range(8):
                    if hs == 1:
                        wj = jnp.take_along_axis(wrow, zeros16 + j, axis=0,
                                                 mode="promise_in_bounds")
                    else:
                        wj = wrow
                    hb[e, pl.ds(j * 16, 16)] = hb[e, pl.ds(j * 16, 16)] * wj
                return c2

            lax.fori_loop(0, BB, edge, 0, unroll=2)

        for b, s in ((0, sets[0]), (1, sets[1])):
            start(i1_cps(b, s))
            start(i2_cps(b, s))
        for b, s in ((0, sets[0]), (1, sets[1])):
            wait(i1_cps(b, s))
            start(g_cps(s))

        def iter_pair(t, carry):
            a = 2 * t
            for off in (0, 1):
                s = sets[off]
                b = a + off
                nxt = b + 2
                wait(g_cps(s))

                @pl.when(nxt < NB)
                def _():
                    start(i1_cps(nxt, s))

                compute(s)
                wait(i2_cps(b, s))
                w_start(s)
            for off in (0, 1):
                s = sets[off]
                b = a + off
                nxt = b + 2
                w_wait(s)

                @pl.when(nxt < NB)
                def _():
                    start(i2_cps(nxt, s))
                    wait(i1_cps(nxt, s))
                    start(g_cps(s))

            return carry

        lax.fori_loop(0, NB // 2, iter_pair, 0)

        sA = sets[0]
        wait(g_cps(sA))
        compute(sA)
        wait(i2_cps(NB - 1, sA))
        w_start(sA)
        w_wait(sA)
        plsc.subcore_barrier()

        def export(c, carry):
            r0 = tid * ZROWS_PT + c * BB
            pltpu.sync_copy(dsh.at[pl.ds(r0, BB)], ex0)
            pltpu.sync_copy(ex0, dp_r.at[cid, pl.ds(r0, BB)])
            pltpu.sync_copy(osh.at[pl.ds(r0, BB)], hb0)
            pltpu.sync_copy(hb0, op_r.at[cid, pl.ds(r0, BB)])
            return carry

        lax.fori_loop(0, ZROWS_PT // BB, export, 0)

    call = pl.kernel(
        body,
        out_type=(jax.ShapeDtypeStruct((NC, NPAD, TW), f32),
                  jax.ShapeDtypeStruct((NC, NPAD, F), f32)),
        mesh=_MESH,
        compiler_params=_SC_PARAMS,
        scratch_types=[
            pltpu.VMEM((BB,), i32), pltpu.VMEM((BB,), i32),
            pltpu.VMEM((BB,), i32),
            pltpu.VMEM((BB, TW), f32), pltpu.VMEM((BB, TW), f32),
            pltpu.VMEM((BB, TW), f32), pltpu.VMEM((BB, F), f32),
            pltpu.VMEM((BB,), i32), pltpu.VMEM((BB,), i32),
            pltpu.VMEM((BB,), i32),
            pltpu.VMEM((BB, TW), f32), pltpu.VMEM((BB, TW), f32),
            pltpu.VMEM((BB, TW), f32), pltpu.VMEM((BB, F), f32),
            pltpu.SemaphoreType.DMA, pltpu.SemaphoreType.DMA,
            pltpu.SemaphoreType.DMA, pltpu.SemaphoreType.DMA,
            pltpu.SemaphoreType.DMA, pltpu.SemaphoreType.DMA,
            pltpu.SemaphoreType.DMA, pltpu.SemaphoreType.DMA,
            pltpu.VMEM_SHARED((NPAD, TW), f32),
            pltpu.VMEM_SHARED((NPAD, F), f32),
        ],
    )
    return call(src, dst, S, D, h, Z8, Z128)


def kernel(x, edge_index, W1, att_src1, att_dst1, b1,
           W2, att_src2, att_dst2, b2, Wc1, bc1, Wc2, bc2):
    src = edge_index[0]
    dst = edge_index[1]

    a1s = att_src1.reshape(8, 16)
    a1d = att_dst1.reshape(8, 16)
    eye8 = jnp.eye(8, dtype=f32)
    As1 = jnp.pad(jnp.einsum("hc,hg->hcg", a1s, eye8).reshape(F, 8),
                  ((0, 0), (0, TW - 8)))
    Ad1 = jnp.pad(jnp.einsum("hc,hg->hcg", a1d, eye8).reshape(F, 8),
                  ((0, 0), (0, TW - 8)))
    As2 = jnp.pad(att_src2.reshape(F, 1), ((0, 0), (0, TW - 1)))
    Ad2 = jnp.pad(att_dst2.reshape(F, 1), ((0, 0), (0, TW - 1)))

    Z8 = jnp.zeros((NPAD, TW), f32)
    Z128 = jnp.zeros((NPAD, F), f32)
    b1r = b1.reshape(1, F)
    b2r = b2.reshape(1, F)
    bc1r = bc1.reshape(1, F)
    Wc2p = jnp.pad(Wc2, ((0, 0), (0, F - 2)))
    bc2p = jnp.pad(bc2, (0, F - 2)).reshape(1, F)

    h1, S1, D1 = _tc_prep(x, W1, As1, Ad1)
    dp1, op1 = _sc_edge(src, dst, S1, D1, h1, Z8, Z128, hs=1)
    h2, S2, D2 = _tc_mid(op1, dp1, b1r, W2, As2, Ad2)
    dp2, op2 = _sc_edge(src, dst, S2, D2, h2, Z8, Z128, hs=0)
    emb, logitsp = _tc_final(op2, dp2, b2r, Wc1, bc1r, Wc2p, bc2p)
    return emb, logitsp[:, :2]


# speedup vs baseline: 1.3253x; 1.0015x over previous
import jax
import jax.numpy as jnp
from jax import lax
from jax.experimental import pallas as pl
from jax.experimental.pallas import tpu as pltpu
from jax.experimental.pallas import tpu_sc as plsc

f32 = jnp.float32
i32 = jnp.int32

N = 10000
E = 320000
F = 128
TW = 16
NC, NS = 2, 16
NW = NC * NS
EPW = E // NW
BB = 80
NB = EPW // BB
NPAD = 10240
ZROWS_PT = NPAD // NS

_MESH = plsc.VectorSubcoreMesh(
    core_axis_name="c", subcore_axis_name="s", num_cores=NC, num_subcores=NS)
_SC_PARAMS = pltpu.CompilerParams(
    needs_layout_passes=False, use_tc_tiling_on_sc=False)


def _blk(shape, imap):
    return pl.BlockSpec(shape, imap)


def _tc_prep(xin, W, Asrc, Adst):
    def body(x_r, w_r, as_r, ad_r, h_r, s_r, d_r):
        h = jnp.dot(x_r[...], w_r[...], preferred_element_type=f32)
        h_r[...] = h
        s_r[...] = jnp.dot(h, as_r[...], preferred_element_type=f32, precision=lax.Precision.HIGHEST)
        d_r[...] = jnp.dot(h, ad_r[...], preferred_element_type=f32, precision=lax.Precision.HIGHEST)
    nb = 10
    bn = N // nb
    return pl.pallas_call(
        body,
        grid=(nb,),
        in_specs=[_blk((bn, F), lambda i: (i, 0)),
                  _blk((F, F), lambda i: (0, 0)),
                  _blk((F, TW), lambda i: (0, 0)),
                  _blk((F, TW), lambda i: (0, 0))],
        out_specs=[_blk((bn, F), lambda i: (i, 0)),
                   _blk((bn, TW), lambda i: (i, 0)),
                   _blk((bn, TW), lambda i: (i, 0))],
        out_shape=[jax.ShapeDtypeStruct((N, F), f32),
                   jax.ShapeDtypeStruct((N, TW), f32),
                   jax.ShapeDtypeStruct((N, TW), f32)],
    )(xin, W, Asrc, Adst)


def _tc_mid(oparts, dparts, b1, W2, Asrc, Adst):
    nb = 10
    bn = N // nb

    def body(p_r, dp_r, b_r, w_r, as_r, ad_r, h2_r, s_r, d_r):
        raw = p_r[0] + p_r[1]
        den = dp_r[0] + dp_r[1]
        rd = 1.0 / (den[:, 0:8] + 1e-16)
        rde = jnp.reshape(
            jnp.broadcast_to(rd[:, :, None], (bn, 8, 16)), (bn, F))
        h = jax.nn.relu(raw * rde + b_r[...])
        h2 = jnp.dot(h, w_r[...], preferred_element_type=f32)
        h2_r[...] = h2
        s_r[...] = jnp.dot(h2, as_r[...], preferred_element_type=f32, precision=lax.Precision.HIGHEST)
        d_r[...] = jnp.dot(h2, ad_r[...], preferred_element_type=f32, precision=lax.Precision.HIGHEST)

    return pl.pallas_call(
        body,
        grid=(nb,),
        in_specs=[_blk((NC, bn, F), lambda i: (0, i, 0)),
                  _blk((NC, bn, TW), lambda i: (0, i, 0)),
                  _blk((1, F), lambda i: (0, 0)),
                  _blk((F, F), lambda i: (0, 0)),
                  _blk((F, TW), lambda i: (0, 0)),
                  _blk((F, TW), lambda i: (0, 0))],
        out_specs=[_blk((bn, F), lambda i: (i, 0)),
                   _blk((bn, TW), lambda i: (i, 0)),
                   _blk((bn, TW), lambda i: (i, 0))],
        out_shape=[jax.ShapeDtypeStruct((N, F), f32),
                   jax.ShapeDtypeStruct((N, TW), f32),
                   jax.ShapeDtypeStruct((N, TW), f32)],
    )(oparts, dparts, b1, W2, Asrc, Adst)


def _tc_final(oparts, dparts, b2, Wc1, bc1, Wc2p, bc2p):
    nb = 10
    bn = N // nb

    def body(p_r, dp_r, b_r, w1_r, b1_r, w2_r, b2_r, emb_r, lg_r):
        raw = p_r[0] + p_r[1]
        den = dp_r[0] + dp_r[1]
        rd = 1.0 / (den[:, 0:1] + 1e-16)
        emb = raw * jnp.broadcast_to(rd, (bn, F)) + b_r[...]
        emb_r[...] = emb
        hc = jax.nn.relu(jnp.dot(emb, w1_r[...], preferred_element_type=f32)
                         + b1_r[...])
        lg_r[...] = jnp.dot(hc, w2_r[...], preferred_element_type=f32) + b2_r[...]

    return pl.pallas_call(
        body,
        grid=(nb,),
        in_specs=[_blk((NC, bn, F), lambda i: (0, i, 0)),
                  _blk((NC, bn, TW), lambda i: (0, i, 0)),
                  _blk((1, F), lambda i: (0, 0)),
                  _blk((F, F), lambda i: (0, 0)),
                  _blk((1, F), lambda i: (0, 0)),
                  _blk((F, F), lambda i: (0, 0)),
                  _blk((1, F), lambda i: (0, 0))],
        out_specs=[_blk((bn, F), lambda i: (i, 0)),
                   _blk((bn, F), lambda i: (i, 0))],
        out_shape=[jax.ShapeDtypeStruct((N, F), f32),
                   jax.ShapeDtypeStruct((N, F), f32)],
    )(oparts, dparts, b2, Wc1, bc1, Wc2p, bc2p)


def _sc_edge(src, dst, S, D, h, Z8, Z128, hs):
    def body(src_r, dst_r, s_r, d_r, h_r, z8_r, z128_r, dp_r, op_r,
             si0, di0, dw0, sg0, dg0, ex0, hb0,
             si1, di1, dw1, sg1, dg1, ex1, hb1,
             is0, is1, i2s0, i2s1, gs0, gs1, ws0, ws1, dsh, osh):
        tid = lax.axis_index("s")
        cid = lax.axis_index("c")
        wid = cid * NS + tid
        ebase = wid * EPW
        sets = ((si0, di0, dw0, sg0, dg0, ex0, hb0, is0, i2s0, gs0, ws0),
                (si1, di1, dw1, sg1, dg1, ex1, hb1, is1, i2s1, gs1, ws1))

        def zinit(c, carry):
            r0 = tid * ZROWS_PT + c * BB
            pltpu.sync_copy(z8_r.at[pl.ds(r0, BB)], ex0)
            pltpu.sync_copy(ex0, dsh.at[pl.ds(r0, BB)])
            pltpu.sync_copy(z128_r.at[pl.ds(r0, BB)], hb0)
            pltpu.sync_copy(hb0, osh.at[pl.ds(r0, BB)])
            return carry

        lax.fori_loop(0, ZROWS_PT // BB, zinit, 0)
        pltpu.sync_copy(z8_r.at[pl.ds(0, BB)], ex0)
        pltpu.sync_copy(z8_r.at[pl.ds(0, BB)], ex1)
        plsc.subcore_barrier()

        rowpat = lax.iota(i32, 16) >> 3
        cols8 = lax.iota(i32, 16) & 7
        zeros16 = jnp.zeros((16,), i32)
        hcols = cols8 if hs == 1 else zeros16

        def i1_cps(b, s):
            base = ebase + b * BB
            return (pltpu.make_async_copy(src_r.at[pl.ds(base, BB)], s[0], s[7]),
                    pltpu.make_async_copy(dst_r.at[pl.ds(base, BB)], s[1], s[7]))

        def i2_cps(b, s):
            base = ebase + b * BB
            return (pltpu.make_async_copy(dst_r.at[pl.ds(base, BB)], s[2], s[8]),)

        def g_cps(s):
            return (pltpu.make_async_copy(s_r.at[s[0]], s[3], s[9]),
                    pltpu.make_async_copy(d_r.at[s[1]], s[4], s[9]),
                    pltpu.make_async_copy(h_r.at[s[0]], s[6], s[9]))

        def w_start(s):
            pltpu.async_copy(s[5], dsh.at[s[2]], s[10], add=True)
            pltpu.async_copy(s[6], osh.at[s[2]], s[10], add=True)

        def w_wait(s):
            pltpu.make_async_copy(s[5], dsh.at[s[2]], s[10]).wait()
            pltpu.make_async_copy(s[6], osh.at[s[2]], s[10]).wait()

        def start(cps):
            for c in cps:
                c.start()

        def wait(cps):
            for c in cps:
                c.wait()

        def compute(s):
            sg, dg, exb, hb = s[3], s[4], s[5], s[6]

            def pair(k, c2):
                rows = rowpat + 2 * k
                a = (plsc.load_gather(sg, [rows, cols8])
                     + plsc.load_gather(dg, [rows, cols8]))
                a = jnp.where(a > 0, a, 0.2 * a)
                plsc.store_scatter(exb, [rows, cols8], jnp.exp(a))
                return c2

            lax.fori_loop(0, BB // 2, pair, 0, unroll=4)

            def edge(e, c2):
                esplat = zeros16 + e
                wrow = plsc.load_gather(exb, [esplat, hcols])
                for j in range(8):
                    if hs == 1:
                        wj = jnp.take_along_axis(wrow, zeros16 + j, axis=0,
                                                 mode="promise_in_bounds")
                    else:
                        wj = wrow
                    hb[e, pl.ds(j * 16, 16)] = hb[e, pl.ds(j * 16, 16)] * wj
                return c2

            lax.fori_loop(0, BB, edge, 0, unroll=2)

        for b, s in ((0, sets[0]), (1, sets[1])):
            start(i1_cps(b, s))
            start(i2_cps(b, s))
        for b, s in ((0, sets[0]), (1, sets[1])):
            wait(i1_cps(b, s))
            start(g_cps(s))

        def iter_pair(t, carry):
            a = 2 * t
            for off in (0, 1):
                s = sets[off]
                b = a + off
                nxt = b + 2
                wait(g_cps(s))

                @pl.when(nxt < NB)
                def _():
                    start(i1_cps(nxt, s))

                compute(s)
                wait(i2_cps(b, s))
                w_start(s)
            for off in (0, 1):
                s = sets[off]
                b = a + off
                nxt = b + 2
                w_wait(s)

                @pl.when(nxt < NB)
                def _():
                    start(i2_cps(nxt, s))
                    wait(i1_cps(nxt, s))
                    start(g_cps(s))

            return carry

        lax.fori_loop(0, NB // 2, iter_pair, 0)

        sA = sets[0]
        wait(g_cps(sA))
        compute(sA)
        wait(i2_cps(NB - 1, sA))
        w_start(sA)
        w_wait(sA)
        plsc.subcore_barrier()

        def export(c, carry):
            r0 = tid * ZROWS_PT + c * BB
            pltpu.sync_copy(dsh.at[pl.ds(r0, BB)], ex0)
            pltpu.sync_copy(ex0, dp_r.at[cid, pl.ds(r0, BB)])
            pltpu.sync_copy(osh.at[pl.ds(r0, BB)], hb0)
            pltpu.sync_copy(hb0, op_r.at[cid, pl.ds(r0, BB)])
            return carry

        lax.fori_loop(0, ZROWS_PT // BB, export, 0)

    call = pl.kernel(
        body,
        out_type=(jax.ShapeDtypeStruct((NC, NPAD, TW), f32),
                  jax.ShapeDtypeStruct((NC, NPAD, F), f32)),
        mesh=_MESH,
        compiler_params=_SC_PARAMS,
        scratch_types=[
            pltpu.VMEM((BB,), i32), pltpu.VMEM((BB,), i32),
            pltpu.VMEM((BB,), i32),
            pltpu.VMEM((BB, TW), f32), pltpu.VMEM((BB, TW), f32),
            pltpu.VMEM((BB, TW), f32), pltpu.VMEM((BB, F), f32),
            pltpu.VMEM((BB,), i32), pltpu.VMEM((BB,), i32),
            pltpu.VMEM((BB,), i32),
            pltpu.VMEM((BB, TW), f32), pltpu.VMEM((BB, TW), f32),
            pltpu.VMEM((BB, TW), f32), pltpu.VMEM((BB, F), f32),
            pltpu.SemaphoreType.DMA, pltpu.SemaphoreType.DMA,
            pltpu.SemaphoreType.DMA, pltpu.SemaphoreType.DMA,
            pltpu.SemaphoreType.DMA, pltpu.SemaphoreType.DMA,
            pltpu.SemaphoreType.DMA, pltpu.SemaphoreType.DMA,
            pltpu.VMEM_SHARED((NPAD, TW), f32),
            pltpu.VMEM_SHARED((NPAD, F), f32),
        ],
    )
    return call(src, dst, S, D, h, Z8, Z128)


def kernel(x, edge_index, W1, att_src1, att_dst1, b1,
           W2, att_src2, att_dst2, b2, Wc1, bc1, Wc2, bc2):
    src = edge_index[0]
    dst = edge_index[1]

    a1s = att_src1.reshape(8, 16)
    a1d = att_dst1.reshape(8, 16)
    eye8 = jnp.eye(8, dtype=f32)
    As1 = jnp.pad(jnp.einsum("hc,hg->hcg", a1s, eye8).reshape(F, 8),
                  ((0, 0), (0, TW - 8)))
    Ad1 = jnp.pad(jnp.einsum("hc,hg->hcg", a1d, eye8).reshape(F, 8),
                  ((0, 0), (0, TW - 8)))
    As2 = jnp.pad(att_src2.reshape(F, 1), ((0, 0), (0, TW - 1)))
    Ad2 = jnp.pad(att_dst2.reshape(F, 1), ((0, 0), (0, TW - 1)))

    Z8 = jnp.zeros((NPAD, TW), f32)
    Z128 = jnp.zeros((NPAD, F), f32)
    b1r = b1.reshape(1, F)
    b2r = b2.reshape(1, F)
    bc1r = bc1.reshape(1, F)
    Wc2p = jnp.pad(Wc2, ((0, 0), (0, F - 2)))
    bc2p = jnp.pad(bc2, (0, F - 2)).reshape(1, F)

    h1, S1, D1 = _tc_prep(x, W1, As1, Ad1)
    dp1, op1 = _sc_edge(src, dst, S1, D1, h1, Z8, Z128, hs=1)
    h2, S2, D2 = _tc_mid(op1, dp1, b1r, W2, As2, Ad2)
    dp2, op2 = _sc_edge(src, dst, S2, D2, h2, Z8, Z128, hs=0)
    emb, logitsp = _tc_final(op2, dp2, b2r, Wc1, bc1r, Wc2p, bc2p)
    return emb, logitsp[:, :2]
